# Initial kernel scaffold; baseline (speedup 1.0000x reference)
import jax
import jax.numpy as jnp
from jax.experimental import pallas as pl


def kernel(x, Wr, W1, V1, W2):
    raise NotImplementedError("write your pallas kernel here")


# speedup vs baseline: 1.4491x; 1.4491x over previous
import functools

import jax
import jax.numpy as jnp
from jax.experimental import pallas as pl
from jax.experimental.pallas import tpu as pltpu

H = 1024
F = 2048
E = 16
S = 2048

SBLK = 512
FBLK = 1024
NSB = S // SBLK
NFB = F // FBLK


def _router_body(x_ref, wr_ref, we_ref):
    x = x_ref[...]
    wr = wr_ref[...]
    logits = jnp.dot(x, wr, preferred_element_type=jnp.float32)
    m = jnp.max(logits, axis=-1, keepdims=True)
    p = jnp.exp(logits - m)
    probs = p / jnp.sum(p, axis=-1, keepdims=True)
    ids = jax.lax.broadcasted_iota(jnp.int32, probs.shape, 1)
    m1 = jnp.max(probs, axis=-1, keepdims=True)
    i1 = jnp.min(jnp.where(probs == m1, ids, E), axis=-1, keepdims=True)
    masked = jnp.where(ids == i1, -1.0, probs)
    m2 = jnp.max(masked, axis=-1, keepdims=True)
    i2 = jnp.min(jnp.where(masked == m2, ids, E), axis=-1, keepdims=True)
    denom = m1 + m2
    we = jnp.where(ids == i1, m1, 0.0) + jnp.where(ids == i2, m2, 0.0)
    we_ref[...] = (we / denom).T[:, None, :]


def _moe_body(we_ref, x_ref, w1_ref, v1_ref, w2_ref, out_ref, xb_ref):
    e = pl.program_id(0)
    fb = pl.program_id(1)
    sb = pl.program_id(2)

    @pl.when(jnp.logical_and(jnp.logical_and(e == 0, fb == 0), sb == 0))
    def _():
        xb_ref[...] = x_ref[...].astype(jnp.bfloat16)

    xs = xb_ref[pl.ds(sb * SBLK, SBLK), :]
    w1b = w1_ref[0].astype(jnp.bfloat16)
    v1b = v1_ref[0].astype(jnp.bfloat16)
    w2b = w2_ref[0].astype(jnp.bfloat16)
    a = jnp.dot(xs, w1b, preferred_element_type=jnp.float32)
    c = jnp.dot(xs, v1b, preferred_element_type=jnp.float32)
    h = (a * jax.nn.sigmoid(a) * c).astype(jnp.bfloat16)
    y = jnp.dot(h, w2b, preferred_element_type=jnp.float32)
    w = we_ref[0, 0, pl.ds(sb * SBLK, SBLK)]
    contrib = y * w[:, None]

    @pl.when(jnp.logical_and(e == 0, fb == 0))
    def _():
        out_ref[pl.ds(sb * SBLK, SBLK), :] = contrib

    @pl.when(jnp.logical_not(jnp.logical_and(e == 0, fb == 0)))
    def _():
        out_ref[pl.ds(sb * SBLK, SBLK), :] += contrib


@functools.partial(jax.jit, static_argnames=("interpret",))
def kernel(x, Wr, W1, V1, W2, interpret=False):
    b, s, hd = x.shape
    xf = x.reshape(s, hd)

    we = pl.pallas_call(
        _router_body,
        out_shape=jax.ShapeDtypeStruct((E, 1, S), jnp.float32),
        interpret=interpret,
    )(xf, Wr)

    out = pl.pallas_call(
        _moe_body,
        grid=(E, NFB, NSB),
        in_specs=[
            pl.BlockSpec((1, 1, S), lambda e, fb, sb: (e, 0, 0)),
            pl.BlockSpec((S, H), lambda e, fb, sb: (0, 0)),
            pl.BlockSpec((1, H, FBLK), lambda e, fb, sb: (e, 0, fb)),
            pl.BlockSpec((1, H, FBLK), lambda e, fb, sb: (e, 0, fb)),
            pl.BlockSpec((1, FBLK, H), lambda e, fb, sb: (e, fb, 0)),
        ],
        out_specs=pl.BlockSpec((S, H), lambda e, fb, sb: (0, 0)),
        out_shape=jax.ShapeDtypeStruct((S, H), jnp.float32),
        scratch_shapes=[pltpu.VMEM((S, H), jnp.bfloat16)],
        interpret=interpret,
    )(we, xf, W1, V1, W2)

    return out.reshape(b, s, hd)


# speedup vs baseline: 2.7435x; 1.8933x over previous
import functools

import jax
import jax.numpy as jnp
from jax import lax
from jax.experimental import pallas as pl
from jax.experimental.pallas import tpu as pltpu
from jax.experimental.pallas import tpu_sc as plsc

H = 1024
F = 2048
E = 16
S = 2048
K = 2
P = K * S
TILE = 128
NT = 48
PAD = NT * TILE

NC, NS = 2, 16
NW = NC * NS
PPW = P // NW
CH = 32
NCH = PPW // CH


def _router_body(x_ref, wr_ref, inv_ref, w_ref, gidx_ref):
    x = x_ref[...]
    logits = jnp.dot(x, wr_ref[...], preferred_element_type=jnp.float32)
    m = jnp.max(logits, axis=-1, keepdims=True)
    ex = jnp.exp(logits - m)
    probs = ex / jnp.sum(ex, axis=-1, keepdims=True)
    ids = lax.broadcasted_iota(jnp.int32, probs.shape, 1)
    m1 = jnp.max(probs, axis=-1, keepdims=True)
    i1 = jnp.min(jnp.where(probs == m1, ids, E), axis=-1, keepdims=True)
    masked = jnp.where(ids == i1, -1.0, probs)
    m2 = jnp.max(masked, axis=-1, keepdims=True)
    i2 = jnp.min(jnp.where(masked == m2, ids, E), axis=-1, keepdims=True)
    denom = m1 + m2
    w_ref[...] = jnp.concatenate([m1 / denom, m2 / denom], axis=1)

    e_all = jnp.concatenate([i1, i2], axis=0)
    onehot = (e_all == lax.broadcasted_iota(jnp.int32, (P, E), 1)
              ).astype(jnp.float32)
    c = onehot
    d = 1
    while d < P:
        c = c + jnp.concatenate(
            [jnp.zeros((d, E), jnp.float32), c[:-d, :]], axis=0)
        d *= 2
    counts = c[P - 1:P, :]
    tiles = jnp.floor((counts + (TILE - 1)) * (1.0 / TILE))
    incl = tiles
    d = 1
    while d < E:
        incl = incl + jnp.concatenate(
            [jnp.zeros((1, d), jnp.float32), incl[:, :-d]], axis=1)
        d *= 2
    tb_excl = incl - tiles
    base = tb_excl * TILE
    slot = jnp.sum((base + c - 1.0) * onehot, axis=1, keepdims=True)
    inv_ref[...] = slot.astype(jnp.int32)

    jj = lax.broadcasted_iota(jnp.int32, (NT, E), 0).astype(jnp.float32)
    g = jnp.sum((incl <= jj).astype(jnp.float32), axis=1, keepdims=True)
    gidx_ref[...] = jnp.minimum(g, float(E - 1)).astype(jnp.int32)


def _gmm_body(gidx_ref, xs_ref, w1_ref, v1_ref, w2_ref, ys_ref):
    xt = xs_ref[...]
    a = jnp.dot(xt, w1_ref[0], preferred_element_type=jnp.float32)
    b = jnp.dot(xt, v1_ref[0], preferred_element_type=jnp.float32)
    h = a * jax.nn.sigmoid(a) * b
    ys_ref[...] = jnp.dot(h, w2_ref[0], preferred_element_type=jnp.float32)


def _combine_body(w_ref, y0_ref, y1_ref, out_ref):
    w0 = w_ref[:, 0:1]
    w1 = w_ref[:, 1:2]
    out_ref[...] = w0 * y0_ref[...] + w1 * y1_ref[...]


def _dispatch_x_body(x_hbm, inv_hbm, xs_hbm, idx_v, rows_v, sem):
    wid = lax.axis_index("s") * NC + lax.axis_index("c")
    pltpu.sync_copy(inv_hbm.at[pl.ds(wid * NCH, NCH)], idx_v)
    for c in range(NCH):
        t0 = lax.rem(wid * PPW + c * CH, S)
        pltpu.sync_copy(x_hbm.at[pl.ds(t0, CH)], rows_v)
        pltpu.async_copy(rows_v, xs_hbm.at[idx_v.at[c]], sem).wait()


def _gather_y_body(ys_hbm, inv_hbm, ysg_hbm, idx_v, rows_v, sem):
    wid = lax.axis_index("s") * NC + lax.axis_index("c")
    pltpu.sync_copy(inv_hbm.at[pl.ds(wid * NCH, NCH)], idx_v)
    for c in range(NCH):
        pltpu.async_copy(ys_hbm.at[idx_v.at[c]], rows_v, sem).wait()
        pltpu.sync_copy(rows_v, ysg_hbm.at[pl.ds(wid * PPW + c * CH, CH)])


def _sc_call(body, out_rows):
    mesh = plsc.VectorSubcoreMesh(
        core_axis_name="c", subcore_axis_name="s", num_cores=NC)
    return pl.kernel(
        body,
        mesh=mesh,
        out_type=jax.ShapeDtypeStruct((out_rows, H), jnp.float32),
        scratch_types=[
            pltpu.VMEM((NCH, CH), jnp.int32),
            pltpu.VMEM((CH, H), jnp.float32),
            pltpu.SemaphoreType.DMA,
        ],
    )


@jax.jit
def kernel(x, Wr, W1, V1, W2):
    b, s, hd = x.shape
    xf = x.reshape(s, hd)

    inv, wcomb, gidx = pl.pallas_call(
        _router_body,
        out_shape=(
            jax.ShapeDtypeStruct((P, 1), jnp.int32),
            jax.ShapeDtypeStruct((S, K), jnp.float32),
            jax.ShapeDtypeStruct((NT, 1), jnp.int32),
        ),
    )(xf, Wr)

    inv_chunks = inv.reshape(NW * NCH, CH)
    gidx_flat = gidx.reshape(NT)

    xs = _sc_call(_dispatch_x_body, PAD)(xf, inv_chunks)

    ys = pl.pallas_call(
        _gmm_body,
        grid_spec=pltpu.PrefetchScalarGridSpec(
            num_scalar_prefetch=1,
            grid=(NT,),
            in_specs=[
                pl.BlockSpec((TILE, H), lambda j, g: (j, 0)),
                pl.BlockSpec((1, H, F), lambda j, g: (g[j], 0, 0)),
                pl.BlockSpec((1, H, F), lambda j, g: (g[j], 0, 0)),
                pl.BlockSpec((1, F, H), lambda j, g: (g[j], 0, 0)),
            ],
            out_specs=pl.BlockSpec((TILE, H), lambda j, g: (j, 0)),
        ),
        out_shape=jax.ShapeDtypeStruct((PAD, H), jnp.float32),
    )(gidx_flat, xs, W1, V1, W2)

    ysg = _sc_call(_gather_y_body, P)(ys, inv_chunks)

    out = pl.pallas_call(
        _combine_body,
        grid=(1,),
        in_specs=[
            pl.BlockSpec((S, K), lambda i: (0, 0)),
            pl.BlockSpec((S, H), lambda i: (0, 0)),
            pl.BlockSpec((S, H), lambda i: (1, 0)),
        ],
        out_specs=pl.BlockSpec((S, H), lambda i: (0, 0)),
        out_shape=jax.ShapeDtypeStruct((S, H), jnp.float32),
    )(wcomb, ysg, ysg)

    return out.reshape(b, s, hd)
